# baseline (speedup 1.0000x reference)
import functools

import jax
import jax.numpy as jnp
from jax import lax
from jax.experimental import pallas as pl
from jax.experimental.pallas import tpu as pltpu
from jax.experimental.pallas import tpu_sc as plsc

D = 64
R = 16
SCALING = 2.0

NC = 2
NS = 16
NW = NC * NS


def _comb_body(tt_ref, a_ref, comb_ref):
    comb_ref[:, 0:D] = tt_ref[...].T
    comb_ref[:, D:D + R] = a_ref[...].T
    comb_ref[:, D + R:] = jnp.zeros(
        (comb_ref.shape[0], 128 - D - R), jnp.float32)


def _make_sc_gather(n_idx: int, ch: int):
    bpw = n_idx // NW
    nchunk = bpw // ch
    assert bpw % ch == 0 and n_idx % NW == 0

    mesh = plsc.VectorSubcoreMesh(core_axis_name="c", subcore_axis_name="s")

    @functools.partial(
        pl.kernel,
        mesh=mesh,
        compiler_params=pltpu.CompilerParams(use_tc_tiling_on_sc=False),
        out_type=jax.ShapeDtypeStruct((n_idx, 128), jnp.float32),
        scratch_types=[
            pltpu.VMEM((ch,), jnp.int32),
            pltpu.VMEM((ch, 128), jnp.float32),
            pltpu.SemaphoreType.DMA,
        ],
    )
    def sc_gather(comb_hbm, idx_hbm, g_hbm, idx_v, rows_v, sem):
        wid = lax.axis_index("s") * NC + lax.axis_index("c")

        @pl.loop(0, nchunk)
        def _(c):
            off = wid * bpw + c * ch
            pltpu.sync_copy(idx_hbm.at[pl.ds(off, ch)], idx_v)
            pltpu.async_copy(comb_hbm.at[idx_v], rows_v, sem).wait()
            pltpu.sync_copy(rows_v, g_hbm.at[pl.ds(off, ch)])

    return sc_gather


def _fuse_body(g_ref, b_ref, o_ref):
    g = g_ref[...]
    base_t = g[:, 0:D].T
    aa = g[:, D:D + R]
    delta_t = lax.dot_general(
        b_ref[...], aa,
        dimension_numbers=(((1,), (1,)), ((), ())),
        preferred_element_type=jnp.float32,
        precision=lax.Precision.HIGHEST,
    )
    o_ref[0] = base_t + delta_t * SCALING


def kernel(x, table, lora_A, lora_B):
    bsz, seq = x.shape
    n = bsz * seq
    vocab = table.shape[0]
    idx = jnp.transpose(x).reshape(n).astype(jnp.int32)

    tbl_t = table.T
    wblk = 4096
    comb = pl.pallas_call(
        _comb_body,
        grid=(pl.cdiv(vocab, wblk),),
        compiler_params=pltpu.CompilerParams(
            dimension_semantics=("parallel",)),
        in_specs=[
            pl.BlockSpec((D, wblk), lambda i: (0, i)),
            pl.BlockSpec((R, wblk), lambda i: (0, i)),
        ],
        out_specs=pl.BlockSpec((wblk, 128), lambda i: (i, 0)),
        out_shape=jax.ShapeDtypeStruct((vocab, 128), jnp.float32),
    )(tbl_t, lora_A)

    g = _make_sc_gather(n, 640)(comb, idx)

    out3 = pl.pallas_call(
        _fuse_body,
        grid=(seq,),
        compiler_params=pltpu.CompilerParams(
            dimension_semantics=("parallel",)),
        in_specs=[
            pl.BlockSpec((bsz, 128), lambda l: (l, 0)),
            pl.BlockSpec((D, R), lambda l: (0, 0)),
        ],
        out_specs=pl.BlockSpec((1, D, bsz), lambda l: (l, 0, 0)),
        out_shape=jax.ShapeDtypeStruct((seq, D, bsz), jnp.float32),
    )(g, lora_B)
    return jnp.transpose(out3, (2, 0, 1))


# speedup vs baseline: 1.0965x; 1.0965x over previous
import functools

import jax
import jax.numpy as jnp
from jax import lax
from jax.experimental import pallas as pl
from jax.experimental.pallas import tpu as pltpu
from jax.experimental.pallas import tpu_sc as plsc

D = 64
R = 16
SCALING = 2.0

NC = 2
NS = 16
NW = NC * NS


def _comb_body(tt_ref, a_ref, comb_ref):
    comb_ref[:, 0:D] = tt_ref[...].T
    comb_ref[:, D:D + R] = a_ref[...].T
    comb_ref[:, D + R:] = jnp.zeros(
        (comb_ref.shape[0], 128 - D - R), jnp.float32)


def _make_sc_gather(n_idx: int, ch: int):
    bpw = n_idx // NW
    nchunk = bpw // ch
    assert bpw % ch == 0 and n_idx % NW == 0

    mesh = plsc.VectorSubcoreMesh(core_axis_name="c", subcore_axis_name="s")

    @functools.partial(
        pl.kernel,
        mesh=mesh,
        compiler_params=pltpu.CompilerParams(use_tc_tiling_on_sc=False),
        out_type=jax.ShapeDtypeStruct((n_idx, 128), jnp.float32),
        scratch_types=[
            pltpu.VMEM((ch,), jnp.int32),
            pltpu.VMEM((ch, 128), jnp.float32),
            pltpu.SemaphoreType.DMA,
        ],
    )
    def sc_gather(comb_hbm, idx_hbm, g_hbm, idx_v, rows_v, sem):
        wid = lax.axis_index("s") * NC + lax.axis_index("c")

        @pl.loop(0, nchunk)
        def _(c):
            off = wid * bpw + c * ch
            pltpu.sync_copy(idx_hbm.at[pl.ds(off, ch)], idx_v)
            pltpu.async_copy(comb_hbm.at[idx_v], rows_v, sem).wait()
            pltpu.sync_copy(rows_v, g_hbm.at[pl.ds(off, ch)])

    return sc_gather


def _fuse_body(g_ref, b_ref, o_ref):
    g = g_ref[...]
    base_t = g[:, 0:D].T
    aa = g[:, D:D + R]
    delta_t = lax.dot_general(
        b_ref[...], aa,
        dimension_numbers=(((1,), (1,)), ((), ())),
        preferred_element_type=jnp.float32,
        precision=lax.Precision.HIGHEST,
    )
    o_ref[0] = base_t + delta_t * SCALING


def kernel(x, table, lora_A, lora_B):
    bsz, seq = x.shape
    n = bsz * seq
    vocab = table.shape[0]
    idx = jnp.transpose(x).reshape(n).astype(jnp.int32)

    tbl_t = table.T
    wblk = 16384
    comb = pl.pallas_call(
        _comb_body,
        grid=(pl.cdiv(vocab, wblk),),
        compiler_params=pltpu.CompilerParams(
            dimension_semantics=("parallel",)),
        in_specs=[
            pl.BlockSpec((D, wblk), lambda i: (0, i)),
            pl.BlockSpec((R, wblk), lambda i: (0, i)),
        ],
        out_specs=pl.BlockSpec((wblk, 128), lambda i: (i, 0)),
        out_shape=jax.ShapeDtypeStruct((vocab, 128), jnp.float32),
    )(tbl_t, lora_A)

    g = _make_sc_gather(n, 640)(comb, idx)

    out3 = pl.pallas_call(
        _fuse_body,
        grid=(seq,),
        compiler_params=pltpu.CompilerParams(
            dimension_semantics=("parallel",)),
        in_specs=[
            pl.BlockSpec((bsz, 128), lambda l: (l, 0)),
            pl.BlockSpec((D, R), lambda l: (0, 0)),
        ],
        out_specs=pl.BlockSpec((1, D, bsz), lambda l: (l, 0, 0)),
        out_shape=jax.ShapeDtypeStruct((seq, D, bsz), jnp.float32),
    )(g, lora_B)
    return jnp.transpose(out3, (2, 0, 1))
